# baseline (speedup 1.0000x reference)
import functools

import jax
import jax.numpy as jnp
from jax import lax
from jax.experimental import pallas as pl
from jax.experimental.pallas import tpu as pltpu
from jax.experimental.pallas import tpu_sc as plsc

VOCAB = 1000000
EMBED = 64
B = 1024
L = 200
OUT = 2
PAD = 0

NC = 2
NS = 16
NW = NC * NS
ROWS_PER_W = B // NW
LPAD = 208
C0 = 128
C1 = L - C0
NLANE = 16
PACK = EMBED // 2
TW = 2048


def _pack_table(vt):
    def tk(vt_ref, out_ref):
        t = jnp.transpose(vt_ref[...])
        lo = lax.convert_element_type(t[:, 0:PACK], jnp.bfloat16)
        hi = lax.convert_element_type(t[:, PACK:EMBED], jnp.bfloat16)
        lo32 = lax.convert_element_type(
            lax.bitcast_convert_type(lo, jnp.uint16), jnp.uint32)
        hi32 = lax.convert_element_type(
            lax.bitcast_convert_type(hi, jnp.uint16), jnp.uint32)
        word = lax.bitwise_or(lax.shift_left(hi32, jnp.uint32(16)), lo32)
        out_ref[...] = lax.bitcast_convert_type(word, jnp.float32)

    grid = (VOCAB + TW - 1) // TW
    return pl.pallas_call(
        tk,
        grid=(grid,),
        in_specs=[pl.BlockSpec((EMBED, TW), lambda j: (0, j))],
        out_specs=pl.BlockSpec((TW, PACK), lambda j: (j, 0)),
        out_shape=jax.ShapeDtypeStruct((VOCAB, PACK), jnp.float32),
    )(vt)


def _sc_pool(ids_flat, vectors, w):
    mesh = plsc.VectorSubcoreMesh(core_axis_name="c", subcore_axis_name="s")

    @functools.partial(
        pl.kernel,
        out_type=(
            jax.ShapeDtypeStruct((B, EMBED), jnp.float32),
            jax.ShapeDtypeStruct((B, NLANE), jnp.float32),
        ),
        mesh=mesh,
        compiler_params=pltpu.CompilerParams(
            use_tc_tiling_on_sc=True, needs_layout_passes=False),
        scratch_types=[
            pltpu.VMEM((LPAD,), jnp.int32),
            pltpu.VMEM((LPAD,), jnp.float32),
            pltpu.VMEM((L, PACK), jnp.float32),
            pltpu.VMEM((LPAD,), jnp.float32),
            pltpu.VMEM((ROWS_PER_W, EMBED), jnp.float32),
            pltpu.VMEM((ROWS_PER_W, NLANE), jnp.float32),
            pltpu.SemaphoreType.DMA,
            pltpu.SemaphoreType.DMA,
        ],
    )
    def k(ids_hbm, vec_hbm, w_hbm, pooled_hbm, len_hbm,
          idx_v, wv_v, rows_v, wt_v, pooled_v, len_v, sem, semr):
        wid = lax.axis_index("s") * NC + lax.axis_index("c")
        row0 = wid * ROWS_PER_W
        lanes = lax.iota(jnp.int32, NLANE)

        def body(i, _):
            row = row0 + i
            base = pl.multiple_of(row * L, 8)
            pltpu.sync_copy(ids_hbm.at[pl.ds(base, L)], idx_v.at[pl.ds(0, L)])
            cps = (
                pltpu.async_copy(w_hbm.at[idx_v.at[pl.ds(0, C0)]],
                                 wv_v.at[pl.ds(0, C0)], sem),
                pltpu.async_copy(w_hbm.at[idx_v.at[pl.ds(C0, C1)]],
                                 wv_v.at[pl.ds(C0, C1)], sem),
            )
            for g in range(L // NLANE):
                idg = idx_v[pl.ds(g * NLANE, NLANE)]
                for j in range(NLANE):
                    l = g * NLANE + j
                    pltpu.async_copy(vec_hbm.at[idg[j]], rows_v.at[l], semr)
            idg = idx_v[pl.ds((L // NLANE) * NLANE, NLANE)]
            for j in range(L % NLANE):
                l = (L // NLANE) * NLANE + j
                pltpu.async_copy(vec_hbm.at[idg[j]], rows_v.at[l], semr)

            for cp in cps:
                cp.wait()
            cnt = jnp.zeros((NLANE,), jnp.float32)
            for c in range(LPAD // NLANE):
                ids_c = idx_v[pl.ds(c * NLANE, NLANE)]
                wv_c = wv_v[pl.ds(c * NLANE, NLANE)]
                m = jnp.logical_and(lanes + (c * NLANE) < L, ids_c != PAD)
                sig = 1.0 / (1.0 + jnp.exp(-wv_c))
                wt_v[pl.ds(c * NLANE, NLANE)] = jnp.where(m, sig, 0.0)
                cnt = cnt + jnp.where(m, 1.0, 0.0)
            len_v[i, pl.ds(0, NLANE)] = cnt

            pltpu.make_async_copy(
                vec_hbm.at[pl.ds(0, L)], rows_v, semr).wait()

            def addto(accs, l, s):
                out = list(accs)
                for k in range(PACK // NLANE):
                    word = plsc.bitcast(
                        rows_v[l, pl.ds(k * NLANE, NLANE)], jnp.uint32)
                    flo = plsc.bitcast(
                        lax.shift_left(word, jnp.uint32(16)), jnp.float32)
                    fhi = plsc.bitcast(
                        lax.bitwise_and(word, jnp.uint32(0xFFFF0000)),
                        jnp.float32)
                    out[k] = out[k] + s * flo
                    out[k + PACK // NLANE] = out[k + PACK // NLANE] + s * fhi
                return tuple(out)

            def group_body(g, accs):
                gbase = pl.multiple_of(g * NLANE, NLANE)
                wtg = wt_v[pl.ds(gbase, NLANE)]
                for j in range(NLANE):
                    accs = addto(accs, gbase + j, wtg[j])
                return accs

            accs = lax.fori_loop(
                0, L // NLANE, group_body,
                tuple(jnp.zeros((NLANE,), jnp.float32)
                      for _ in range(EMBED // NLANE)))
            gbase = (L // NLANE) * NLANE
            wtg = wt_v[pl.ds(gbase, NLANE)]
            for j in range(L % NLANE):
                accs = addto(accs, gbase + j, wtg[j])
            for j in range(EMBED // NLANE):
                pooled_v[i, pl.ds(j * NLANE, NLANE)] = accs[j]
            return 0

        lax.fori_loop(0, ROWS_PER_W, body, 0)
        pltpu.sync_copy(pooled_v, pooled_hbm.at[pl.ds(row0, ROWS_PER_W)])
        pltpu.sync_copy(len_v, len_hbm.at[pl.ds(row0, ROWS_PER_W)])

    return k(ids_flat, vectors, w)


def _head(pooled, counts, head_W, head_b):
    def hk(p_ref, l_ref, w_ref, b_ref, log_ref, enc_ref):
        length = jnp.sum(l_ref[...], axis=1, keepdims=True) + 1e-16
        p = p_ref[...] / length
        norm = jnp.sqrt(jnp.sum(p * p, axis=1, keepdims=True))
        enc = p / jnp.maximum(norm, 1e-12)
        enc_ref[...] = enc
        log_ref[...] = (
            jnp.dot(enc, w_ref[...], preferred_element_type=jnp.float32)
            + b_ref[...])

    return pl.pallas_call(
        hk,
        out_shape=(
            jax.ShapeDtypeStruct((B, OUT), jnp.float32),
            jax.ShapeDtypeStruct((B, EMBED), jnp.float32),
        ),
    )(pooled, counts, head_W, head_b)


def kernel(input_ids, vectors, w, head_W, head_b):
    ids_flat = input_ids.reshape(-1).astype(jnp.int32)
    packed = _pack_table(vectors.T)
    pooled, counts = _sc_pool(ids_flat, packed, w)
    logits, encoded = _head(pooled, counts, head_W, head_b.reshape(1, OUT))
    return (logits, encoded)


# speedup vs baseline: 1.5641x; 1.5641x over previous
import functools

import jax
import jax.numpy as jnp
from jax import lax
from jax.experimental import pallas as pl
from jax.experimental.pallas import tpu as pltpu
from jax.experimental.pallas import tpu_sc as plsc

VOCAB = 1000000
EMBED = 64
B = 1024
L = 200
OUT = 2
PAD = 0

NC = 2
NS = 16
NW = NC * NS
ROWS_PER_W = B // NW
LPAD = 208
C0 = 128
C1 = L - C0
NLANE = 16


def _sc_pool(ids_flat, vectors, w):
    mesh = plsc.VectorSubcoreMesh(core_axis_name="c", subcore_axis_name="s")

    @functools.partial(
        pl.kernel,
        out_type=(
            jax.ShapeDtypeStruct((B, EMBED), jnp.float32),
            jax.ShapeDtypeStruct((B, NLANE), jnp.float32),
        ),
        mesh=mesh,
        compiler_params=pltpu.CompilerParams(use_tc_tiling_on_sc=True),
        scratch_types=[
            pltpu.VMEM((ROWS_PER_W * L + NLANE,), jnp.int32),
            pltpu.VMEM((2, LPAD), jnp.float32),
            pltpu.VMEM((2, L, EMBED), jnp.float32),
            pltpu.VMEM((2, LPAD), jnp.float32),
            pltpu.VMEM((ROWS_PER_W, EMBED), jnp.float32),
            pltpu.VMEM((ROWS_PER_W, NLANE), jnp.float32),
            pltpu.SemaphoreType.DMA,
            pltpu.SemaphoreType.DMA,
            pltpu.SemaphoreType.DMA,
            pltpu.SemaphoreType.DMA,
        ],
    )
    def k(ids_hbm, vec_hbm, w_hbm, pooled_hbm, len_hbm,
          idx_v, wv_v, rows_v, wt_v, pooled_v, len_v,
          semw0, semw1, semr0, semr1):
        wid = lax.axis_index("s") * NC + lax.axis_index("c")
        row0 = wid * ROWS_PER_W
        lanes = lax.iota(jnp.int32, NLANE)
        semw = (semw0, semw1)
        semr = (semr0, semr1)

        pltpu.sync_copy(ids_hbm.at[pl.ds(pl.multiple_of(row0 * L, 8),
                                         ROWS_PER_W * L)],
                        idx_v.at[pl.ds(0, ROWS_PER_W * L)])

        def fire(i, p):
            ib = pl.multiple_of(i * L, 8)
            pltpu.async_copy(w_hbm.at[idx_v.at[pl.ds(ib, C0)]],
                             wv_v.at[p, pl.ds(0, C0)], semw[p])
            pltpu.async_copy(w_hbm.at[idx_v.at[pl.ds(ib + C0, C1)]],
                             wv_v.at[p, pl.ds(C0, C1)], semw[p])
            for g in range(L // NLANE):
                idg = idx_v[pl.ds(ib + g * NLANE, NLANE)]
                for j in range(NLANE):
                    pltpu.async_copy(vec_hbm.at[idg[j]],
                                     rows_v.at[p, g * NLANE + j], semr[p])
            idg = idx_v[pl.ds(ib + (L // NLANE) * NLANE, NLANE)]
            for j in range(L % NLANE):
                pltpu.async_copy(vec_hbm.at[idg[j]],
                                 rows_v.at[p, (L // NLANE) * NLANE + j],
                                 semr[p])

        def consume(i, p):
            ib = pl.multiple_of(i * L, 8)
            pltpu.make_async_copy(w_hbm.at[pl.ds(0, C0)],
                                  wv_v.at[p, pl.ds(0, C0)], semw[p]).wait()
            pltpu.make_async_copy(w_hbm.at[pl.ds(0, C1)],
                                  wv_v.at[p, pl.ds(C0, C1)], semw[p]).wait()
            cnt = jnp.zeros((NLANE,), jnp.float32)
            for c in range(LPAD // NLANE):
                ids_c = idx_v[pl.ds(ib + c * NLANE, NLANE)]
                wv_c = wv_v[p, pl.ds(c * NLANE, NLANE)]
                m = jnp.logical_and(lanes + (c * NLANE) < L, ids_c != PAD)
                sig = 1.0 / (1.0 + jnp.exp(-wv_c))
                wt_v[p, pl.ds(c * NLANE, NLANE)] = jnp.where(m, sig, 0.0)
                cnt = cnt + jnp.where(m, 1.0, 0.0)
            len_v[i, pl.ds(0, NLANE)] = cnt

            pltpu.make_async_copy(
                vec_hbm.at[pl.ds(0, L)], rows_v.at[p], semr[p]).wait()

            def addto(accs, l, s):
                return tuple(
                    accs[k] + s * rows_v[p, l, pl.ds(k * NLANE, NLANE)]
                    for k in range(EMBED // NLANE))

            def group_body(g, accs):
                gbase = pl.multiple_of(g * NLANE, NLANE)
                wtg = wt_v[p, pl.ds(gbase, NLANE)]
                for j in range(NLANE):
                    accs = addto(accs, gbase + j, wtg[j])
                return accs

            accs = lax.fori_loop(
                0, L // NLANE, group_body,
                tuple(jnp.zeros((NLANE,), jnp.float32)
                      for _ in range(EMBED // NLANE)))
            gbase = (L // NLANE) * NLANE
            wtg = wt_v[p, pl.ds(gbase, NLANE)]
            for j in range(L % NLANE):
                accs = addto(accs, gbase + j, wtg[j])
            for j in range(EMBED // NLANE):
                pooled_v[i, pl.ds(j * NLANE, NLANE)] = accs[j]

        fire(0, 0)

        def ubody(u, _):
            te = 2 * u
            fire(te + 1, 1)
            consume(te, 0)
            fire(te + 2, 0)
            consume(te + 1, 1)
            return 0

        lax.fori_loop(0, ROWS_PER_W // 2 - 1, ubody, 0)
        fire(ROWS_PER_W - 1, 1)
        consume(ROWS_PER_W - 2, 0)
        consume(ROWS_PER_W - 1, 1)

        pltpu.sync_copy(pooled_v, pooled_hbm.at[pl.ds(row0, ROWS_PER_W)])
        pltpu.sync_copy(len_v, len_hbm.at[pl.ds(row0, ROWS_PER_W)])

    return k(ids_flat, vectors, w)


def _head(pooled, counts, head_W, head_b):
    def hk(p_ref, l_ref, w_ref, b_ref, log_ref, enc_ref):
        length = jnp.sum(l_ref[...], axis=1, keepdims=True) + 1e-16
        p = p_ref[...] / length
        norm = jnp.sqrt(jnp.sum(p * p, axis=1, keepdims=True))
        enc = p / jnp.maximum(norm, 1e-12)
        enc_ref[...] = enc
        log_ref[...] = (
            jnp.dot(enc, w_ref[...], preferred_element_type=jnp.float32)
            + b_ref[...])

    return pl.pallas_call(
        hk,
        out_shape=(
            jax.ShapeDtypeStruct((B, OUT), jnp.float32),
            jax.ShapeDtypeStruct((B, EMBED), jnp.float32),
        ),
    )(pooled, counts, head_W, head_b)


def kernel(input_ids, vectors, w, head_W, head_b):
    ids_flat = input_ids.reshape(-1).astype(jnp.int32)
    pooled, counts = _sc_pool(ids_flat, vectors, w)
    logits, encoded = _head(pooled, counts, head_W, head_b.reshape(1, OUT))
    return (logits, encoded)
